# baseline (speedup 1.0000x reference)
import jax
import jax.numpy as jnp
from jax.experimental import pallas as pl

POOL = 20
TOPK = 5
NTOK = 197
DIM = 768
NLAYERS = 12
COLS = NTOK + 2 * POOL
KPAD = 128


BB = 8


def _body(x_ref, kp_ref, mask_ref, dacc_ref):
    b = pl.program_id(0)

    @pl.when(b == 0)
    def _():
        dacc_ref[...] = jnp.zeros((8, 128), jnp.float32)

    for i in range(BB):
        _one_batch(x_ref, kp_ref, mask_ref, dacc_ref, i)


def _one_batch(x_ref, kp_ref, mask_ref, dacc_ref, i):
    xb = x_ref[i]
    xn = xb / jnp.maximum(jnp.sqrt(jnp.sum(xb * xb, axis=1, keepdims=True)), 1e-12)
    kp = kp_ref[...]
    kn = kp / jnp.maximum(jnp.sqrt(jnp.sum(kp * kp, axis=1, keepdims=True)), 1e-12)
    sim = jax.lax.dot_general(
        xn, kn, (((1,), (1,)), ((), ())), preferred_element_type=jnp.float32
    )

    r = jax.lax.broadcasted_iota(jnp.int32, (NTOK, KPAD), 0)
    c = jax.lax.broadcasted_iota(jnp.int32, (NTOK, KPAD), 1)
    valid = ((r == 0) & (c >= 1) & (c < 1 + POOL)) | (
        (r != 0) & (c >= 1 + POOL) & (c < 1 + 2 * POOL)
    )
    simv = jnp.where(valid, sim, -2.0)

    ind = jnp.zeros((NTOK, KPAD), jnp.float32)
    ssum = jnp.zeros((NTOK, 1), jnp.float32)
    for _ in range(TOPK):
        m = jnp.max(simv, axis=1, keepdims=True)
        first = jnp.min(jnp.where(simv == m, c, KPAD), axis=1, keepdims=True)
        onehot = c == first
        ind = jnp.where(onehot, 1.0, ind)
        ssum = ssum + m
        simv = jnp.where(onehot, -3.0, simv)

    mask128 = jnp.where((c >= 1) & (c < 1 + 2 * POOL), ind, 1.0)
    tile = jnp.concatenate(
        [mask128, jnp.full((NTOK, COLS - KPAD), 1.0, jnp.float32)], axis=1
    )
    mask_ref[i] = jnp.broadcast_to(tile[None], (NLAYERS, NTOK, COLS))

    rr = jax.lax.broadcasted_iota(jnp.int32, (NTOK, 1), 0)
    cls_sum = jnp.sum(jnp.where(rr == 0, ssum, 0.0))
    img_sum = jnp.sum(jnp.where(rr == 0, 0.0, ssum))
    ar = jax.lax.broadcasted_iota(jnp.int32, (8, 128), 0)
    ac = jax.lax.broadcasted_iota(jnp.int32, (8, 128), 1)
    part = jnp.where((ar == 0) & (ac == 0), cls_sum, 0.0) + jnp.where(
        (ar == 0) & (ac == 1), img_sum, 0.0
    )
    dacc_ref[...] += part


def kernel(x, keys_cls, keys_image, layer):
    B = x.shape[0]
    kc = jnp.take(keys_cls, layer, axis=0)
    ki = jnp.take(keys_image, layer, axis=0)
    kp = (
        jnp.zeros((KPAD, DIM), jnp.float32)
        .at[1 : 1 + POOL]
        .set(kc)
        .at[1 + POOL : 1 + 2 * POOL]
        .set(ki)
    )

    mask, dacc = pl.pallas_call(
        _body,
        grid=(B // BB,),
        in_specs=[
            pl.BlockSpec((BB, NTOK, DIM), lambda b: (b, 0, 0)),
            pl.BlockSpec((KPAD, DIM), lambda b: (0, 0)),
        ],
        out_specs=[
            pl.BlockSpec((BB, NLAYERS, NTOK, COLS), lambda b: (b, 0, 0, 0)),
            pl.BlockSpec((8, 128), lambda b: (0, 0)),
        ],
        out_shape=[
            jax.ShapeDtypeStruct((B, NLAYERS, NTOK, COLS), jnp.float32),
            jax.ShapeDtypeStruct((8, 128), jnp.float32),
        ],
    )(x, kp)

    dist = (1.0 - dacc[0, 0] / (B * TOPK)) + (
        1.0 - dacc[0, 1] / (B * (NTOK - 1) * TOPK)
    )
    return (mask, dist)


# speedup vs baseline: 1.0604x; 1.0604x over previous
import jax
import jax.numpy as jnp
from jax.experimental import pallas as pl

POOL = 20
TOPK = 5
NTOK = 197
DIM = 768
NLAYERS = 12
COLS = NTOK + 2 * POOL
KPAD = 128


BB = 4


def _body(x_ref, kp_ref, mask_ref, dacc_ref):
    b = pl.program_id(0)

    @pl.when(b == 0)
    def _():
        dacc_ref[...] = jnp.zeros((8, 128), jnp.float32)

    for i in range(BB):
        _one_batch(x_ref, kp_ref, mask_ref, dacc_ref, i)


def _one_batch(x_ref, kp_ref, mask_ref, dacc_ref, i):
    xb = x_ref[i]
    xn = xb / jnp.maximum(jnp.sqrt(jnp.sum(xb * xb, axis=1, keepdims=True)), 1e-12)
    kp = kp_ref[...]
    kn = kp / jnp.maximum(jnp.sqrt(jnp.sum(kp * kp, axis=1, keepdims=True)), 1e-12)
    sim = jax.lax.dot_general(
        xn, kn, (((1,), (1,)), ((), ())), preferred_element_type=jnp.float32
    )

    r = jax.lax.broadcasted_iota(jnp.int32, (NTOK, KPAD), 0)
    c = jax.lax.broadcasted_iota(jnp.int32, (NTOK, KPAD), 1)
    valid = ((r == 0) & (c >= 1) & (c < 1 + POOL)) | (
        (r != 0) & (c >= 1 + POOL) & (c < 1 + 2 * POOL)
    )
    simv = jnp.where(valid, sim, -2.0)

    ind = jnp.zeros((NTOK, KPAD), jnp.float32)
    ssum = jnp.zeros((NTOK, 1), jnp.float32)
    for _ in range(TOPK):
        m = jnp.max(simv, axis=1, keepdims=True)
        first = jnp.min(jnp.where(simv == m, c, KPAD), axis=1, keepdims=True)
        onehot = c == first
        ind = jnp.where(onehot, 1.0, ind)
        ssum = ssum + m
        simv = jnp.where(onehot, -3.0, simv)

    mask128 = jnp.where((c >= 1) & (c < 1 + 2 * POOL), ind, 1.0)
    tile = jnp.concatenate(
        [mask128, jnp.full((NTOK, COLS - KPAD), 1.0, jnp.float32)], axis=1
    )
    mask_ref[i] = jnp.broadcast_to(tile[None], (NLAYERS, NTOK, COLS))

    rr = jax.lax.broadcasted_iota(jnp.int32, (NTOK, 1), 0)
    cls_sum = jnp.sum(jnp.where(rr == 0, ssum, 0.0))
    img_sum = jnp.sum(jnp.where(rr == 0, 0.0, ssum))
    ar = jax.lax.broadcasted_iota(jnp.int32, (8, 128), 0)
    ac = jax.lax.broadcasted_iota(jnp.int32, (8, 128), 1)
    part = jnp.where((ar == 0) & (ac == 0), cls_sum, 0.0) + jnp.where(
        (ar == 0) & (ac == 1), img_sum, 0.0
    )
    dacc_ref[...] += part


def kernel(x, keys_cls, keys_image, layer):
    B = x.shape[0]
    kc = jnp.take(keys_cls, layer, axis=0)
    ki = jnp.take(keys_image, layer, axis=0)
    kp = (
        jnp.zeros((KPAD, DIM), jnp.float32)
        .at[1 : 1 + POOL]
        .set(kc)
        .at[1 + POOL : 1 + 2 * POOL]
        .set(ki)
    )

    mask, dacc = pl.pallas_call(
        _body,
        grid=(B // BB,),
        in_specs=[
            pl.BlockSpec((BB, NTOK, DIM), lambda b: (b, 0, 0)),
            pl.BlockSpec((KPAD, DIM), lambda b: (0, 0)),
        ],
        out_specs=[
            pl.BlockSpec((BB, NLAYERS, NTOK, COLS), lambda b: (b, 0, 0, 0)),
            pl.BlockSpec((8, 128), lambda b: (0, 0)),
        ],
        out_shape=[
            jax.ShapeDtypeStruct((B, NLAYERS, NTOK, COLS), jnp.float32),
            jax.ShapeDtypeStruct((8, 128), jnp.float32),
        ],
    )(x, kp)

    dist = (1.0 - dacc[0, 0] / (B * TOPK)) + (
        1.0 - dacc[0, 1] / (B * (NTOK - 1) * TOPK)
    )
    return (mask, dist)
